# baseline (speedup 1.0000x reference)
import functools

import jax
import jax.numpy as jnp
from jax import lax
from jax.experimental import pallas as pl
from jax.experimental.pallas import tpu as pltpu
from jax.experimental.pallas import tpu_sc as plsc

N_SUB = 16
GROUP = 512
FDIM = 128
NEDGE = 131072
NNODES = N_SUB * GROUP
A_SIZE = NNODES * GROUP
WIN = 128
HALF_WORDS = A_SIZE // 2
TILE_WORDS = HALF_WORDS // 16
ZCHUNK = 8192


ECHUNK = NEDGE // 16
NWIN = ECHUNK // WIN


def _sc_build_adj(adj):
    mesh = plsc.VectorSubcoreMesh(core_axis_name="core", subcore_axis_name="subcore")

    @functools.partial(
        pl.kernel,
        out_type=jax.ShapeDtypeStruct((A_SIZE,), jnp.float32),
        mesh=mesh,
        compiler_params=pltpu.CompilerParams(needs_layout_passes=False),
        scratch_types=[
            pltpu.VMEM((ZCHUNK,), jnp.float32),
            pltpu.VMEM((WIN,), jnp.float32),
            pltpu.VMEM((ECHUNK,), jnp.int32),
            pltpu.VMEM((ECHUNK,), jnp.int32),
            pltpu.VMEM((ECHUNK + 16,), jnp.int32),
            pltpu.VMEM((NWIN, WIN), jnp.int32),
            pltpu.SemaphoreType.DMA,
            pltpu.SemaphoreType.DMA,
        ],
    )
    def build(adj_hbm, a_hbm, zbuf, ones, srcv, dstv, cbuf, idx2,
              sem, semz):
        cid = lax.axis_index("core")
        sid = lax.axis_index("subcore")

        ebase = sid * ECHUNK
        cp_s = pltpu.async_copy(adj_hbm.at[0, pl.ds(ebase, ECHUNK)], srcv, sem)
        cp_d = pltpu.async_copy(adj_hbm.at[1, pl.ds(ebase, ECHUNK)], dstv, sem)

        with jax.named_scope("zfill"):
            zero16 = jnp.zeros((16,), jnp.float32)

            @pl.loop(0, ZCHUNK, step=64)
            def _(i):
                zbuf[pl.ds(i, 16)] = zero16
                zbuf[pl.ds(i + 16, 16)] = zero16
                zbuf[pl.ds(i + 32, 16)] = zero16
                zbuf[pl.ds(i + 48, 16)] = zero16

            @pl.loop(0, WIN, step=16)
            def _(i):
                ones[pl.ds(i, 16)] = jnp.full((16,), 1.0, jnp.float32)

        base = cid * HALF_WORDS + sid * TILE_WORDS
        zcps = [
            pltpu.async_copy(zbuf, a_hbm.at[pl.ds(base + j * ZCHUNK, ZCHUNK)],
                             semz)
            for j in range(TILE_WORDS // ZCHUNK)
        ]

        with jax.named_scope("edge_wait"):
            cp_s.wait()
            cp_d.wait()

        def cbody(i, off):
            vals = []
            for u in range(4):
                sl = pl.ds(i * 64 + u * 16, 16)
                sv = srcv[sl]
                dv = dstv[sl]
                valid = ((sv >> 9) == (dv >> 9)) & ((sv >> 12) == cid)
                flat = ((sv >> 9) * (GROUP * GROUP)
                        + ((dv >> 7) & 3) * (GROUP * WIN)
                        + (sv & (GROUP - 1)) * WIN
                        + (dv & (WIN - 1)))
                vals.append((valid, flat, jnp.sum(valid.astype(jnp.int32))))
            for valid, flat, pop in vals:
                plsc.store_compressed(cbuf.at[pl.ds(off, 16)], flat, mask=valid)
                off = off + pop
            return off

        with jax.named_scope("compact"):
            cnt = lax.fori_loop(0, ECHUNK // 64, cbody, 0)
        nwin = (cnt + (WIN - 1)) // WIN

        first = plsc.load_gather(cbuf, [jnp.zeros((16,), jnp.int32)])

        def tbody(k, carry):
            sl = pl.ds(k * 16, 16)
            pos = k * 16 + lax.iota(jnp.int32, 16)
            cur = cbuf[sl]
            cbuf[sl] = jnp.where(pos >= cnt, first, cur)
            return carry

        with jax.named_scope("tailfill"):
            lax.fori_loop(cnt // 16, nwin * 8, tbody, 0)

        def copybody(i, carry):
            idx2[i // 8, pl.ds((i % 8) * 16, 16)] = cbuf[pl.ds(i * 16, 16)]
            return carry

        with jax.named_scope("copywin"):
            lax.fori_loop(0, nwin * 8, copybody, 0)

        with jax.named_scope("zero_wait"):
            for z in zcps:
                z.wait()
        with jax.named_scope("barrier"):
            plsc.subcore_barrier()

        def sbody(j, carry):
            pltpu.sync_copy(ones, a_hbm.at[idx2.at[j]])
            return carry

        with jax.named_scope("scatter"):
            lax.fori_loop(0, nwin, sbody, 0)

    return build(adj)


def _tc_softmax_body(x_ref, w_ref, b_ref, s_ref, out_ref):
    x = x_ref[0]
    w = w_ref[...]
    b = b_ref[...]
    xh = x.astype(jnp.bfloat16)
    logits = jnp.dot(xh, w.astype(jnp.bfloat16),
                     preferred_element_type=jnp.float32) + b
    m = jnp.max(logits, axis=1, keepdims=True)
    e = jnp.exp(logits - m)
    s = e / jnp.sum(e, axis=1, keepdims=True)
    sh = s.astype(jnp.bfloat16)
    s_ref[0] = sh
    out_ref[0] = lax.dot_general(
        sh, xh, (((0,), (0,)), ((), ())), preferred_element_type=jnp.float32)


def _tc_softmax(x16, w, b2):
    s16, out = pl.pallas_call(
        _tc_softmax_body,
        grid=(N_SUB,),
        in_specs=[
            pl.BlockSpec((1, GROUP, FDIM), lambda i: (i, 0, 0)),
            pl.BlockSpec((FDIM, GROUP), lambda i: (0, 0)),
            pl.BlockSpec((1, GROUP), lambda i: (0, 0)),
        ],
        out_specs=[
            pl.BlockSpec((1, GROUP, GROUP), lambda i: (i, 0, 0)),
            pl.BlockSpec((1, GROUP, FDIM), lambda i: (i, 0, 0)),
        ],
        out_shape=[
            jax.ShapeDtypeStruct((N_SUB, GROUP, GROUP), jnp.bfloat16),
            jax.ShapeDtypeStruct((N_SUB, GROUP, FDIM), jnp.float32),
        ],
    )(x16, w, b2)
    return s16, out


def _tc_diag_body(a_ref, s_ref, diag_ref):
    sh = s_ref[0]
    s = sh.astype(jnp.float32)
    a = a_ref[0].astype(jnp.bfloat16)
    tmp = jnp.dot(a[0 * GROUP:1 * GROUP], sh[0 * WIN:1 * WIN],
                  preferred_element_type=jnp.float32)
    tmp += jnp.dot(a[1 * GROUP:2 * GROUP], sh[1 * WIN:2 * WIN],
                   preferred_element_type=jnp.float32)
    tmp += jnp.dot(a[2 * GROUP:3 * GROUP], sh[2 * WIN:3 * WIN],
                   preferred_element_type=jnp.float32)
    tmp += jnp.dot(a[3 * GROUP:4 * GROUP], sh[3 * WIN:4 * WIN],
                   preferred_element_type=jnp.float32)
    diag_ref[0] = jnp.sum(s * tmp, axis=0, keepdims=True)


def _tc_diag(a16, s16):
    return pl.pallas_call(
        _tc_diag_body,
        grid=(N_SUB,),
        in_specs=[
            pl.BlockSpec((1, 4 * GROUP, WIN), lambda i: (i, 0, 0)),
            pl.BlockSpec((1, GROUP, GROUP), lambda i: (i, 0, 0)),
        ],
        out_specs=pl.BlockSpec((1, 1, GROUP), lambda i: (i, 0, 0)),
        out_shape=jax.ShapeDtypeStruct((N_SUB, 1, GROUP), jnp.float32),
    )(a16, s16)


def kernel(temporal_graph, temporal_adj, W_pool, b_pool):
    x16 = temporal_graph.reshape(N_SUB, GROUP, FDIM)

    a_flat = _sc_build_adj(temporal_adj.astype(jnp.int32))
    a16 = a_flat.reshape(N_SUB, 4 * GROUP, WIN)

    s16, out = _tc_softmax(x16, W_pool, b_pool.reshape(1, GROUP))
    diag = _tc_diag(a16, s16)

    temporal_pooled = out.reshape(1, NNODES, FDIM)
    new_weights = diag.reshape(NNODES)
    ar = jnp.arange(NNODES, dtype=temporal_adj.dtype)
    new_adj = jnp.stack([ar, ar])
    return (temporal_pooled, new_adj, new_weights)


# speedup vs baseline: 1.0707x; 1.0707x over previous
import functools

import jax
import jax.numpy as jnp
from jax import lax
from jax.experimental import pallas as pl
from jax.experimental.pallas import tpu as pltpu
from jax.experimental.pallas import tpu_sc as plsc

N_SUB = 16
GROUP = 512
FDIM = 128
NEDGE = 131072
NNODES = N_SUB * GROUP
A_SIZE = NNODES * GROUP
WIN = 128
HALF_WORDS = A_SIZE // 2
TILE_WORDS = HALF_WORDS // 16
ZCHUNK = 8192


ECHUNK = NEDGE // 16
NWIN = ECHUNK // WIN


def _sc_build_adj(adj):
    mesh = plsc.VectorSubcoreMesh(core_axis_name="core", subcore_axis_name="subcore")

    @functools.partial(
        pl.kernel,
        out_type=jax.ShapeDtypeStruct((A_SIZE,), jnp.float32),
        mesh=mesh,
        compiler_params=pltpu.CompilerParams(needs_layout_passes=False),
        scratch_types=[
            pltpu.VMEM((ZCHUNK,), jnp.float32),
            pltpu.VMEM((WIN,), jnp.float32),
            pltpu.VMEM((ECHUNK,), jnp.int32),
            pltpu.VMEM((ECHUNK,), jnp.int32),
            pltpu.VMEM((ECHUNK + 16,), jnp.int32),
            pltpu.VMEM((NWIN, WIN), jnp.int32),
            pltpu.SemaphoreType.DMA,
            pltpu.SemaphoreType.DMA,
        ],
    )
    def build(adj_hbm, a_hbm, zbuf, ones, srcv, dstv, cbuf, idx2,
              sem, semz):
        cid = lax.axis_index("core")
        sid = lax.axis_index("subcore")

        ebase = sid * ECHUNK
        cp_s = pltpu.async_copy(adj_hbm.at[0, pl.ds(ebase, ECHUNK)], srcv, sem)
        cp_d = pltpu.async_copy(adj_hbm.at[1, pl.ds(ebase, ECHUNK)], dstv, sem)

        with jax.named_scope("zfill"):
            zero16 = jnp.zeros((16,), jnp.float32)

            @pl.loop(0, ZCHUNK, step=64)
            def _(i):
                zbuf[pl.ds(i, 16)] = zero16
                zbuf[pl.ds(i + 16, 16)] = zero16
                zbuf[pl.ds(i + 32, 16)] = zero16
                zbuf[pl.ds(i + 48, 16)] = zero16

            @pl.loop(0, WIN, step=16)
            def _(i):
                ones[pl.ds(i, 16)] = jnp.full((16,), 1.0, jnp.float32)

        base = cid * HALF_WORDS + sid * TILE_WORDS
        zcps = [
            pltpu.async_copy(zbuf, a_hbm.at[pl.ds(base + j * ZCHUNK, ZCHUNK)],
                             semz)
            for j in range(TILE_WORDS // ZCHUNK)
        ]

        with jax.named_scope("edge_wait"):
            cp_s.wait()
            cp_d.wait()

        def cbody(i, off):
            vals = []
            for u in range(4):
                sl = pl.ds(i * 64 + u * 16, 16)
                sv = srcv[sl]
                dv = dstv[sl]
                valid = ((sv >> 9) == (dv >> 9)) & ((sv >> 12) == cid)
                flat = ((sv >> 9) * (GROUP * GROUP)
                        + ((dv >> 7) & 3) * (GROUP * WIN)
                        + (sv & (GROUP - 1)) * WIN
                        + (dv & (WIN - 1)))
                vals.append((valid, flat, jnp.sum(valid.astype(jnp.int32))))
            for valid, flat, pop in vals:
                plsc.store_compressed(cbuf.at[pl.ds(off, 16)], flat, mask=valid)
                off = off + pop
            return off

        with jax.named_scope("compact"):
            cnt = lax.fori_loop(0, ECHUNK // 64, cbody, 0)
        nwin = (cnt + (WIN - 1)) // WIN

        first = plsc.load_gather(cbuf, [jnp.zeros((16,), jnp.int32)])

        def tbody(k, carry):
            sl = pl.ds(k * 16, 16)
            pos = k * 16 + lax.iota(jnp.int32, 16)
            cur = cbuf[sl]
            cbuf[sl] = jnp.where(pos >= cnt, first, cur)
            return carry

        with jax.named_scope("tailfill"):
            lax.fori_loop(cnt // 16, nwin * 8, tbody, 0)

        def copybody(i, carry):
            idx2[i // 8, pl.ds((i % 8) * 16, 16)] = cbuf[pl.ds(i * 16, 16)]
            return carry

        with jax.named_scope("copywin"):
            lax.fori_loop(0, nwin * 8, copybody, 0)

        with jax.named_scope("zero_wait"):
            for z in zcps:
                z.wait()
        with jax.named_scope("barrier"):
            plsc.subcore_barrier()

        def sbody(j, carry):
            pltpu.sync_copy(ones, a_hbm.at[idx2.at[j]])
            return carry

        with jax.named_scope("scatter"):
            lax.fori_loop(0, nwin, sbody, 0)

    return build(adj)


def _tc_softmax_body(x_ref, w_ref, b_ref, s_ref, out_ref):
    x = x_ref[0]
    w = w_ref[...]
    b = b_ref[...]
    xh = x.astype(jnp.bfloat16)
    logits = jnp.dot(xh, w.astype(jnp.bfloat16),
                     preferred_element_type=jnp.float32) + b
    m = jnp.max(logits, axis=1, keepdims=True)
    e = jnp.exp(logits - m)
    s = e / jnp.sum(e, axis=1, keepdims=True)
    sh = s.astype(jnp.bfloat16)
    s_ref[0] = sh
    out_ref[0] = lax.dot_general(
        sh, xh, (((0,), (0,)), ((), ())), preferred_element_type=jnp.float32)


def _tc_softmax(x16, w, b2):
    s16, out = pl.pallas_call(
        _tc_softmax_body,
        grid=(N_SUB,),
        in_specs=[
            pl.BlockSpec((1, GROUP, FDIM), lambda i: (i, 0, 0)),
            pl.BlockSpec((FDIM, GROUP), lambda i: (0, 0)),
            pl.BlockSpec((1, GROUP), lambda i: (0, 0)),
        ],
        out_specs=[
            pl.BlockSpec((1, GROUP, GROUP), lambda i: (i, 0, 0)),
            pl.BlockSpec((1, GROUP, FDIM), lambda i: (i, 0, 0)),
        ],
        out_shape=[
            jax.ShapeDtypeStruct((N_SUB, GROUP, GROUP), jnp.bfloat16),
            jax.ShapeDtypeStruct((N_SUB, GROUP, FDIM), jnp.float32),
        ],
    )(x16, w, b2)
    return s16, out


def _tc_diag_body(a_ref, s_ref, diag_ref):
    for u in range(2):
        sh = s_ref[u]
        s = sh.astype(jnp.float32)
        a = a_ref[u].astype(jnp.bfloat16)
        tmp = jnp.dot(a[0 * GROUP:1 * GROUP], sh[0 * WIN:1 * WIN],
                      preferred_element_type=jnp.float32)
        tmp += jnp.dot(a[1 * GROUP:2 * GROUP], sh[1 * WIN:2 * WIN],
                       preferred_element_type=jnp.float32)
        tmp += jnp.dot(a[2 * GROUP:3 * GROUP], sh[2 * WIN:3 * WIN],
                       preferred_element_type=jnp.float32)
        tmp += jnp.dot(a[3 * GROUP:4 * GROUP], sh[3 * WIN:4 * WIN],
                       preferred_element_type=jnp.float32)
        diag_ref[u] = jnp.sum(s * tmp, axis=0, keepdims=True)


def _tc_diag(a16, s16):
    return pl.pallas_call(
        _tc_diag_body,
        grid=(N_SUB // 2,),
        in_specs=[
            pl.BlockSpec((2, 4 * GROUP, WIN), lambda i: (i, 0, 0)),
            pl.BlockSpec((2, GROUP, GROUP), lambda i: (i, 0, 0)),
        ],
        out_specs=pl.BlockSpec((2, 1, GROUP), lambda i: (i, 0, 0)),
        out_shape=jax.ShapeDtypeStruct((N_SUB, 1, GROUP), jnp.float32),
    )(a16, s16)


def kernel(temporal_graph, temporal_adj, W_pool, b_pool):
    x16 = temporal_graph.reshape(N_SUB, GROUP, FDIM)

    a_flat = _sc_build_adj(temporal_adj.astype(jnp.int32))
    a16 = a_flat.reshape(N_SUB, 4 * GROUP, WIN)

    s16, out = _tc_softmax(x16, W_pool, b_pool.reshape(1, GROUP))
    diag = _tc_diag(a16, s16)

    temporal_pooled = out.reshape(1, NNODES, FDIM)
    new_weights = diag.reshape(NNODES)
    ar = jnp.arange(NNODES, dtype=temporal_adj.dtype)
    new_adj = jnp.stack([ar, ar])
    return (temporal_pooled, new_adj, new_weights)
